# baseline (speedup 1.0000x reference)
import functools

import jax
import jax.numpy as jnp
from jax import lax
from jax.experimental import pallas as pl
from jax.experimental.pallas import tpu as pltpu
from jax.experimental.pallas import tpu_sc as plsc

_B, _C = 16384, 1000
_K = 12288
_CHI, _IHI, _CLO, _ILO = 125, 128, 8, 128
_BI = 16
_NBLK = _IHI // _BI

_NC, _NS = 2, 16
_NW = _NC * _NS
_PER = _B // _NW

_INT_MIN = -2147483648


def _gather_body(flat_ref, idx_ref, out_ref, idx_v, vals_v, sem):
    wid = lax.axis_index("s") * _NC + lax.axis_index("c")
    base = wid * _PER
    pltpu.sync_copy(idx_ref.at[pl.ds(base, _PER)], idx_v)
    pltpu.async_copy(flat_ref.at[idx_v], vals_v, sem).wait()
    pltpu.sync_copy(vals_v, out_ref.at[pl.ds(base, _PER)])


def _sc_gather(flat, idx):
    mesh = plsc.VectorSubcoreMesh(core_axis_name="c", subcore_axis_name="s")
    run = functools.partial(
        pl.kernel,
        mesh=mesh,
        out_type=jax.ShapeDtypeStruct((_B,), jnp.float32),
        scratch_types=[
            pltpu.VMEM((_PER,), jnp.int32),
            pltpu.VMEM((_PER,), jnp.float32),
            pltpu.SemaphoreType.DMA,
        ],
    )(_gather_body)
    return run(flat, idx)


def _loss_body(x_ref, t_ref, loss_ref):
    x = x_ref[...]
    t = t_ref[...]
    m1 = jnp.max(x, axis=0)
    m = jnp.max(m1, axis=1)
    e = jnp.exp(x - m[None, :, None, :])
    s = jnp.sum(jnp.sum(e, axis=0), axis=1)
    t_hi = (t // _CLO)[None, :, None, :]
    chi = jax.lax.broadcasted_iota(jnp.int32, (_CHI, 1, 1, 1), 0)
    z = jnp.sum(jnp.where(chi == t_hi, x, 0.0), axis=0)
    clo = jax.lax.broadcasted_iota(jnp.int32, (_BI, _CLO, _ILO), 1)
    picked = jnp.sum(jnp.where(clo == (t % _CLO)[:, None, :], z, 0.0), axis=1)
    loss_ref[...] = jnp.log(s) + m - picked


def _topk_body(loss_ref, out_ref):
    x = loss_ref[...]
    bits = jax.lax.bitcast_convert_type(x, jnp.int32)
    key = jnp.where(bits >= 0, bits, bits ^ jnp.int32(0x7FFFFFFF))

    def body(r, T):
        sh = jnp.int32(28) - 4 * r
        digit = jnp.int32(0)
        for j in range(1, 16):
            cand = T | jax.lax.shift_left(jnp.int32(j), sh)
            scand = cand ^ jnp.int32(_INT_MIN)
            cnt = jnp.sum((key >= scand).astype(jnp.int32))
            digit += (cnt >= _K).astype(jnp.int32)
        return T | jax.lax.shift_left(digit, sh)

    T = jax.lax.fori_loop(0, 8, body, jnp.int32(0))
    kth = T ^ jnp.int32(_INT_MIN)

    gt = key > kth
    cnt_gt = jnp.sum(gt.astype(jnp.int32))
    sum_gt = jnp.sum(jnp.where(gt, x, 0.0))
    tval = jnp.max(jnp.where(key == kth, x, -jnp.inf))
    res = (sum_gt + (_K - cnt_gt).astype(jnp.float32) * tval) / _K
    out_ref[...] = jnp.full((1, 1), res, jnp.float32)


def kernel(input, target):
    x4 = input.T.reshape(_CHI, _CLO, _IHI, _ILO).transpose(0, 2, 1, 3)
    flat = x4.reshape(-1)

    t2 = target.astype(jnp.int32).reshape(_IHI, _ILO)

    loss = pl.pallas_call(
        _loss_body,
        grid=(_NBLK,),
        in_specs=[
            pl.BlockSpec((_CHI, _BI, _CLO, _ILO), lambda b: (0, b, 0, 0)),
            pl.BlockSpec((_BI, _ILO), lambda b: (b, 0)),
        ],
        out_specs=pl.BlockSpec((_BI, _ILO), lambda b: (b, 0)),
        out_shape=jax.ShapeDtypeStruct((_IHI, _ILO), jnp.float32),
    )(x4, t2)

    out = pl.pallas_call(
        _topk_body,
        out_shape=jax.ShapeDtypeStruct((1, 1), jnp.float32),
    )(loss)
    return out[0, 0]


# speedup vs baseline: 1.3873x; 1.3873x over previous
import functools

import jax
import jax.numpy as jnp
from jax import lax
from jax.experimental import pallas as pl
from jax.experimental.pallas import tpu as pltpu
from jax.experimental.pallas import tpu_sc as plsc

_B, _C = 16384, 1000
_K = 12288
_CHI, _IHI, _CLO, _ILO = 125, 128, 8, 128
_BI = 16
_NBLK = _IHI // _BI

_NC, _NS = 2, 16
_NW = _NC * _NS
_PER = _B // _NW

_INT_MIN = -2147483648


def _gather_body(flat_ref, idx_ref, out_ref, idx_v, vals_v, sem):
    wid = lax.axis_index("s") * _NC + lax.axis_index("c")
    base = wid * _PER
    pltpu.sync_copy(idx_ref.at[pl.ds(base, _PER)], idx_v)
    pltpu.async_copy(flat_ref.at[idx_v], vals_v, sem).wait()
    pltpu.sync_copy(vals_v, out_ref.at[pl.ds(base, _PER)])


def _sc_gather(flat, idx):
    mesh = plsc.VectorSubcoreMesh(core_axis_name="c", subcore_axis_name="s")
    run = functools.partial(
        pl.kernel,
        mesh=mesh,
        out_type=jax.ShapeDtypeStruct((_B,), jnp.float32),
        scratch_types=[
            pltpu.VMEM((_PER,), jnp.int32),
            pltpu.VMEM((_PER,), jnp.float32),
            pltpu.SemaphoreType.DMA,
        ],
    )(_gather_body)
    return run(flat, idx)


def _loss_body(x_ref, t_ref, loss_ref):
    x = x_ref[...]
    t = t_ref[...]
    m1 = jnp.max(x, axis=0)
    m = jnp.max(m1, axis=1)
    e = jnp.exp(x - m[None, :, None, :])
    s = jnp.sum(jnp.sum(e, axis=0), axis=1)
    cls = jax.lax.broadcasted_iota(jnp.int32, (_CHI, _BI, _CLO, _ILO), 0) * _CLO \
        + jax.lax.broadcasted_iota(jnp.int32, (_CHI, _BI, _CLO, _ILO), 2)
    mask = cls == t[None, :, None, :]
    picked = jnp.sum(jnp.sum(jnp.where(mask, x, 0.0), axis=0), axis=1)
    loss_ref[...] = jnp.log(s) + m - picked


def _topk_body(loss_ref, out_ref):
    x = loss_ref[...]
    bits = jax.lax.bitcast_convert_type(x, jnp.int32)
    key = jnp.where(bits >= 0, bits, bits ^ jnp.int32(0x7FFFFFFF))

    def body(r, T):
        sh = jnp.int32(28) - 4 * r
        digit = jnp.int32(0)
        for j in range(1, 16):
            cand = T | jax.lax.shift_left(jnp.int32(j), sh)
            scand = cand ^ jnp.int32(_INT_MIN)
            cnt = jnp.sum((key >= scand).astype(jnp.int32))
            digit += (cnt >= _K).astype(jnp.int32)
        return T | jax.lax.shift_left(digit, sh)

    T = jax.lax.fori_loop(0, 8, body, jnp.int32(0))
    kth = T ^ jnp.int32(_INT_MIN)

    gt = key > kth
    cnt_gt = jnp.sum(gt.astype(jnp.int32))
    sum_gt = jnp.sum(jnp.where(gt, x, 0.0))
    tval = jnp.max(jnp.where(key == kth, x, -jnp.inf))
    res = (sum_gt + (_K - cnt_gt).astype(jnp.float32) * tval) / _K
    out_ref[...] = jnp.full((1, 1), res, jnp.float32)


def kernel(input, target):
    x4 = input.T.reshape(_CHI, _CLO, _IHI, _ILO).transpose(0, 2, 1, 3)
    flat = x4.reshape(-1)

    t2 = target.astype(jnp.int32).reshape(_IHI, _ILO)

    loss = pl.pallas_call(
        _loss_body,
        grid=(_NBLK,),
        in_specs=[
            pl.BlockSpec((_CHI, _BI, _CLO, _ILO), lambda b: (0, b, 0, 0)),
            pl.BlockSpec((_BI, _ILO), lambda b: (b, 0)),
        ],
        out_specs=pl.BlockSpec((_BI, _ILO), lambda b: (b, 0)),
        out_shape=jax.ShapeDtypeStruct((_IHI, _ILO), jnp.float32),
    )(x4, t2)

    out = pl.pallas_call(
        _topk_body,
        out_shape=jax.ShapeDtypeStruct((1, 1), jnp.float32),
    )(loss)
    return out[0, 0]
